# baseline (speedup 1.0000x reference)
import functools

import jax
import jax.numpy as jnp
from jax import lax
from jax.experimental import pallas as pl
from jax.experimental.pallas import tpu as pltpu
from jax.experimental.pallas import tpu_sc as plsc

N = 10000
E = 320000
HID = 32
NG = 64
NCORES = 2
NSUB = 16
NWORK = NCORES * NSUB
CHUNK = 128
EPW = (E + 2 * NWORK * CHUNK - 1) // (2 * NWORK * CHUNK) * 2 * CHUNK
EPAD = EPW * NWORK
NCHUNK = EPW // CHUNK
ROWS_PER_TILE = (N // NSUB + 16) // 16 * 16
NPAD = ROWS_PER_TILE * NSUB

_mesh = plsc.VectorSubcoreMesh(core_axis_name="c", subcore_axis_name="s",
                               num_cores=NCORES, num_subcores=NSUB)


@functools.partial(
    pl.kernel,
    out_type=jax.ShapeDtypeStruct((NWORK, NPAD), jnp.float32),
    mesh=_mesh,
    compiler_params=pltpu.CompilerParams(needs_layout_passes=False),
    scratch_types=[
        pltpu.VMEM((NPAD,), jnp.float32),
        pltpu.VMEM((EPW,), jnp.int32),
    ],
)
def _sc_deg(dsts_hbm, degp_hbm, deg_v, dst_v):
    c = lax.axis_index("c")
    s = lax.axis_index("s")
    wid = c * NSUB + s
    zero16 = jnp.zeros((16,), jnp.float32)
    ones16 = jnp.ones((16,), jnp.float32)

    def zbody(i, carry):
        deg_v[pl.ds(i * 16, 16)] = zero16
        return carry
    lax.fori_loop(0, NPAD // 16, zbody, 0)

    pltpu.sync_copy(dsts_hbm.at[wid], dst_v)

    def ebody(i, carry):
        vals = dst_v[pl.ds(i * 16, 16)]
        plsc.addupdate_scatter(deg_v, [vals], ones16)
        return carry
    lax.fori_loop(0, EPW // 16, ebody, 0)

    pltpu.sync_copy(deg_v, degp_hbm.at[wid])


@functools.partial(
    pl.kernel,
    out_type=jax.ShapeDtypeStruct((NCORES, NPAD, HID), jnp.float32),
    mesh=_mesh,
    compiler_params=pltpu.CompilerParams(needs_layout_passes=False,
                                         use_tc_tiling_on_sc=False),
    scratch_types=[
        pltpu.VMEM((NCHUNK, CHUNK), jnp.int32),
        pltpu.VMEM((NCHUNK, CHUNK), jnp.int32),
        pltpu.VMEM((2, CHUNK, HID), jnp.float32),
        pltpu.VMEM((ROWS_PER_TILE, HID), jnp.float32),
        pltpu.VMEM_SHARED((NPAD, HID), jnp.float32),
        pltpu.SemaphoreType.DMA,
        pltpu.SemaphoreType.DMA,
    ],
)
def _sc_agg(g_hbm, srcs_hbm, dsts_hbm, parts_hbm,
            src_v, dst_v, rows2, z_v, acc_sh, sem0, sem1):
    c = lax.axis_index("c")
    s = lax.axis_index("s")
    wid = c * NSUB + s
    zero16 = jnp.zeros((16,), jnp.float32)

    def zbody(i, carry):
        z_v[i, pl.ds(0, 16)] = zero16
        z_v[i, pl.ds(16, 16)] = zero16
        return carry
    lax.fori_loop(0, ROWS_PER_TILE, zbody, 0)

    pltpu.sync_copy(z_v, acc_sh.at[pl.ds(s * ROWS_PER_TILE, ROWS_PER_TILE)])
    plsc.subcore_barrier()

    pltpu.sync_copy(srcs_hbm.at[wid], src_v)
    pltpu.sync_copy(dsts_hbm.at[wid], dst_v)

    half = NCHUNK // 2
    pltpu.async_copy(g_hbm.at[src_v.at[0]], rows2.at[0], sem0)

    def ebody(i, carry):
        j0 = 2 * i
        pltpu.make_async_copy(g_hbm.at[src_v.at[j0]], rows2.at[0], sem0).wait()
        pltpu.async_copy(g_hbm.at[src_v.at[j0 + 1]], rows2.at[1], sem1)
        pltpu.sync_copy(rows2.at[0], acc_sh.at[dst_v.at[j0]], add=True)
        pltpu.make_async_copy(g_hbm.at[src_v.at[j0 + 1]], rows2.at[1],
                              sem1).wait()

        @pl.when(i < half - 1)
        def _prefetch():
            pltpu.async_copy(g_hbm.at[src_v.at[j0 + 2]], rows2.at[0], sem0)

        pltpu.sync_copy(rows2.at[1], acc_sh.at[dst_v.at[j0 + 1]], add=True)
        return carry
    lax.fori_loop(0, half, ebody, 0)

    plsc.subcore_barrier()
    pltpu.sync_copy(acc_sh.at[pl.ds(s * ROWS_PER_TILE, ROWS_PER_TILE)],
                    parts_hbm.at[c, pl.ds(s * ROWS_PER_TILE, ROWS_PER_TILE)])


def _tc1_body(x_ref, w_ref, degp_ref, h_ref, g_ref, d_ref):
    degp = degp_ref[...]
    ones = jnp.ones((NWORK, 1), jnp.float32)
    deg = lax.dot_general(degp, ones, (((0,), (0,)), ((), ())),
                          preferred_element_type=jnp.float32)
    d = lax.rsqrt(deg[:N] + 1.0)
    h = jnp.dot(x_ref[...], w_ref[...], preferred_element_type=jnp.float32)
    h_ref[...] = h
    g_ref[...] = h * d
    d_ref[...] = d


def _tc_mid_body(parts_ref, h_ref, d_ref, b_ref, w_ref, hn_ref, gn_ref):
    d = d_ref[...]
    agg = parts_ref[0, :N, :] + parts_ref[1, :N, :]
    z = jnp.maximum(d * agg + (d * d) * h_ref[...] + b_ref[...], 0.0)
    hn = jnp.dot(z, w_ref[...], preferred_element_type=jnp.float32)
    hn_ref[...] = hn
    gn_ref[...] = hn * d


def _tc_final_body(parts_ref, h_ref, d_ref, b_ref, batch_ref, wl_ref, bl_ref,
                   out_ref):
    d = d_ref[...]
    agg = parts_ref[0, :N, :] + parts_ref[1, :N, :]
    z = d * agg + (d * d) * h_ref[...] + b_ref[...]
    gid = lax.broadcasted_iota(jnp.int32, (NG, N), 0)
    onehot = (gid == batch_ref[...]).astype(jnp.float32)
    sums = jnp.dot(onehot, z, preferred_element_type=jnp.float32)
    counts = jnp.sum(onehot, axis=1, keepdims=True)
    pooled = sums / jnp.maximum(counts, 1.0)
    out_ref[...] = (jnp.dot(pooled, wl_ref[...],
                            preferred_element_type=jnp.float32) + bl_ref[...])


_tc1 = pl.pallas_call(
    _tc1_body,
    out_shape=(jax.ShapeDtypeStruct((N, HID), jnp.float32),
               jax.ShapeDtypeStruct((N, HID), jnp.float32),
               jax.ShapeDtypeStruct((N, 1), jnp.float32)),
)

_tc_mid = pl.pallas_call(
    _tc_mid_body,
    out_shape=(jax.ShapeDtypeStruct((N, HID), jnp.float32),
               jax.ShapeDtypeStruct((N, HID), jnp.float32)),
)

_tc_final = pl.pallas_call(
    _tc_final_body,
    out_shape=jax.ShapeDtypeStruct((NG, 16), jnp.float32),
)


def kernel(x, edge_index, batch, W1, b1, W2, b2, W3, b3, Wl, bl):
    src = edge_index[0].astype(jnp.int32)
    dst = edge_index[1].astype(jnp.int32)
    npad = EPAD - E
    src_p = jnp.concatenate([src, jnp.zeros((npad,), jnp.int32)])
    dst_p = jnp.concatenate([dst, jnp.full((npad,), N, jnp.int32)])
    srcs = src_p.reshape(NWORK, NCHUNK, CHUNK)
    dsts = dst_p.reshape(NWORK, NCHUNK, CHUNK)
    dsts_flat = dst_p.reshape(NWORK, EPW)

    degp = _sc_deg(dsts_flat)
    h1, g1, d = _tc1(x.astype(jnp.float32), W1, degp)
    p1 = _sc_agg(g1, srcs, dsts)
    h2, g2 = _tc_mid(p1, h1, d, b1.reshape(1, HID), W2)
    p2 = _sc_agg(g2, srcs, dsts)
    h3, g3 = _tc_mid(p2, h2, d, b2.reshape(1, HID), W3)
    p3 = _sc_agg(g3, srcs, dsts)
    out = _tc_final(p3, h3, d, b3.reshape(1, HID),
                    batch.reshape(1, N).astype(jnp.int32), Wl,
                    bl.reshape(1, 16))
    return out


# speedup vs baseline: 1.9857x; 1.9857x over previous
import functools

import jax
import jax.numpy as jnp
from jax import lax
from jax.experimental import pallas as pl
from jax.experimental.pallas import tpu as pltpu
from jax.experimental.pallas import tpu_sc as plsc

N = 10000
E = 320000
HID = 32
NG = 64
NCORES = 2
NSUB = 16
NWORK = NCORES * NSUB
CHUNK = 128
EPW = (E + 2 * NWORK * CHUNK - 1) // (2 * NWORK * CHUNK) * 2 * CHUNK
EPAD = EPW * NWORK
NCHUNK = EPW // CHUNK
ROWS_PER_TILE = (N // NSUB + 16) // 16 * 16
NPAD = ROWS_PER_TILE * NSUB

_mesh = plsc.VectorSubcoreMesh(core_axis_name="c", subcore_axis_name="s",
                               num_cores=NCORES, num_subcores=NSUB)


@functools.partial(
    pl.kernel,
    out_type=jax.ShapeDtypeStruct((NWORK, NPAD), jnp.float32),
    mesh=_mesh,
    compiler_params=pltpu.CompilerParams(needs_layout_passes=False),
    scratch_types=[
        pltpu.VMEM((NPAD,), jnp.float32),
        pltpu.VMEM((EPW,), jnp.int32),
    ],
)
def _sc_deg(dsts_hbm, degp_hbm, deg_v, dst_v):
    c = lax.axis_index("c")
    s = lax.axis_index("s")
    wid = c * NSUB + s
    zero16 = jnp.zeros((16,), jnp.float32)
    ones16 = jnp.ones((16,), jnp.float32)

    def zbody(i, carry):
        deg_v[pl.ds(i * 16, 16)] = zero16
        return carry
    lax.fori_loop(0, NPAD // 16, zbody, 0)

    pltpu.sync_copy(dsts_hbm.at[wid], dst_v)

    def ebody(i, carry):
        vals = dst_v[pl.ds(i * 16, 16)]
        plsc.addupdate_scatter(deg_v, [vals], ones16)
        return carry
    lax.fori_loop(0, EPW // 16, ebody, 0)

    pltpu.sync_copy(deg_v, degp_hbm.at[wid])


@functools.partial(
    pl.kernel,
    out_type=jax.ShapeDtypeStruct((NCORES, NPAD, HID), jnp.float32),
    mesh=_mesh,
    compiler_params=pltpu.CompilerParams(needs_layout_passes=False,
                                         use_tc_tiling_on_sc=False),
    scratch_types=[
        pltpu.VMEM((NCHUNK, CHUNK), jnp.int32),
        pltpu.VMEM((NCHUNK, CHUNK), jnp.int32),
        pltpu.VMEM((2, CHUNK, HID), jnp.float32),
        pltpu.VMEM((ROWS_PER_TILE, HID), jnp.float32),
        pltpu.VMEM_SHARED((NPAD, HID), jnp.float32),
        pltpu.VMEM_SHARED((N, HID), jnp.float32),
        pltpu.SemaphoreType.DMA,
        pltpu.SemaphoreType.DMA,
    ],
)
def _sc_agg(g_hbm, srcs_hbm, dsts_hbm, parts_hbm,
            src_v, dst_v, rows2, z_v, acc_sh, g_sh, sem0, sem1):
    c = lax.axis_index("c")
    s = lax.axis_index("s")
    wid = c * NSUB + s
    zero16 = jnp.zeros((16,), jnp.float32)

    def zbody(i, carry):
        z_v[i, pl.ds(0, 16)] = zero16
        z_v[i, pl.ds(16, 16)] = zero16
        return carry
    lax.fori_loop(0, ROWS_PER_TILE, zbody, 0)

    pltpu.sync_copy(z_v, acc_sh.at[pl.ds(s * ROWS_PER_TILE, ROWS_PER_TILE)])
    pltpu.sync_copy(g_hbm.at[pl.ds(s * (N // NSUB), N // NSUB)],
                    g_sh.at[pl.ds(s * (N // NSUB), N // NSUB)])
    plsc.subcore_barrier()

    pltpu.sync_copy(srcs_hbm.at[wid], src_v)
    pltpu.sync_copy(dsts_hbm.at[wid], dst_v)

    half = NCHUNK // 2
    pltpu.async_copy(g_sh.at[src_v.at[0]], rows2.at[0], sem0)

    def ebody(i, carry):
        j0 = 2 * i
        pltpu.make_async_copy(g_sh.at[src_v.at[j0]], rows2.at[0], sem0).wait()
        pltpu.async_copy(g_sh.at[src_v.at[j0 + 1]], rows2.at[1], sem1)
        pltpu.sync_copy(rows2.at[0], acc_sh.at[dst_v.at[j0]], add=True)
        pltpu.make_async_copy(g_sh.at[src_v.at[j0 + 1]], rows2.at[1],
                              sem1).wait()

        @pl.when(i < half - 1)
        def _prefetch():
            pltpu.async_copy(g_sh.at[src_v.at[j0 + 2]], rows2.at[0], sem0)

        pltpu.sync_copy(rows2.at[1], acc_sh.at[dst_v.at[j0 + 1]], add=True)
        return carry
    lax.fori_loop(0, half, ebody, 0)

    plsc.subcore_barrier()
    pltpu.sync_copy(acc_sh.at[pl.ds(s * ROWS_PER_TILE, ROWS_PER_TILE)],
                    parts_hbm.at[c, pl.ds(s * ROWS_PER_TILE, ROWS_PER_TILE)])


def _tc1_body(x_ref, w_ref, degp_ref, h_ref, g_ref, d_ref):
    degp = degp_ref[...]
    ones = jnp.ones((NWORK, 1), jnp.float32)
    deg = lax.dot_general(degp, ones, (((0,), (0,)), ((), ())),
                          preferred_element_type=jnp.float32)
    d = lax.rsqrt(deg[:N] + 1.0)
    h = jnp.dot(x_ref[...], w_ref[...], preferred_element_type=jnp.float32)
    h_ref[...] = h
    g_ref[...] = h * d
    d_ref[...] = d


def _tc_mid_body(parts_ref, h_ref, d_ref, b_ref, w_ref, hn_ref, gn_ref):
    d = d_ref[...]
    agg = parts_ref[0, :N, :] + parts_ref[1, :N, :]
    z = jnp.maximum(d * agg + (d * d) * h_ref[...] + b_ref[...], 0.0)
    hn = jnp.dot(z, w_ref[...], preferred_element_type=jnp.float32)
    hn_ref[...] = hn
    gn_ref[...] = hn * d


def _tc_final_body(parts_ref, h_ref, d_ref, b_ref, batch_ref, wl_ref, bl_ref,
                   out_ref):
    d = d_ref[...]
    agg = parts_ref[0, :N, :] + parts_ref[1, :N, :]
    z = d * agg + (d * d) * h_ref[...] + b_ref[...]
    gid = lax.broadcasted_iota(jnp.int32, (NG, N), 0)
    onehot = (gid == batch_ref[...]).astype(jnp.float32)
    sums = jnp.dot(onehot, z, preferred_element_type=jnp.float32)
    counts = jnp.sum(onehot, axis=1, keepdims=True)
    pooled = sums / jnp.maximum(counts, 1.0)
    out_ref[...] = (jnp.dot(pooled, wl_ref[...],
                            preferred_element_type=jnp.float32) + bl_ref[...])


_tc1 = pl.pallas_call(
    _tc1_body,
    out_shape=(jax.ShapeDtypeStruct((N, HID), jnp.float32),
               jax.ShapeDtypeStruct((N, HID), jnp.float32),
               jax.ShapeDtypeStruct((N, 1), jnp.float32)),
)

_tc_mid = pl.pallas_call(
    _tc_mid_body,
    out_shape=(jax.ShapeDtypeStruct((N, HID), jnp.float32),
               jax.ShapeDtypeStruct((N, HID), jnp.float32)),
)

_tc_final = pl.pallas_call(
    _tc_final_body,
    out_shape=jax.ShapeDtypeStruct((NG, 16), jnp.float32),
)


def kernel(x, edge_index, batch, W1, b1, W2, b2, W3, b3, Wl, bl):
    src = edge_index[0].astype(jnp.int32)
    dst = edge_index[1].astype(jnp.int32)
    npad = EPAD - E
    src_p = jnp.concatenate([src, jnp.zeros((npad,), jnp.int32)])
    dst_p = jnp.concatenate([dst, jnp.full((npad,), N, jnp.int32)])
    srcs = src_p.reshape(NWORK, NCHUNK, CHUNK)
    dsts = dst_p.reshape(NWORK, NCHUNK, CHUNK)
    dsts_flat = dst_p.reshape(NWORK, EPW)

    degp = _sc_deg(dsts_flat)
    h1, g1, d = _tc1(x.astype(jnp.float32), W1, degp)
    p1 = _sc_agg(g1, srcs, dsts)
    h2, g2 = _tc_mid(p1, h1, d, b1.reshape(1, HID), W2)
    p2 = _sc_agg(g2, srcs, dsts)
    h3, g3 = _tc_mid(p2, h2, d, b2.reshape(1, HID), W3)
    p3 = _sc_agg(g3, srcs, dsts)
    out = _tc_final(p3, h3, d, b3.reshape(1, HID),
                    batch.reshape(1, N).astype(jnp.int32), Wl,
                    bl.reshape(1, 16))
    return out
